# Initial kernel scaffold; baseline (speedup 1.0000x reference)
import jax
import jax.numpy as jnp
from jax.experimental import pallas as pl


def kernel(input_values, masking, gamma, beta, W, codebook):
    raise NotImplementedError("write your pallas kernel here")


# speedup vs baseline: 5.1549x; 5.1549x over previous
import jax
import jax.numpy as jnp
from jax.experimental import pallas as pl

_B, _T, _D = 2, 2048, 768
_Q, _K = 64, 512
_TT = 512
_NB = (_B * _T) // _TT


def _vq_body(x_ref, gamma_ref, beta_ref, wt_ref, cb_ref, cbt_ref,
             xln_ref, lab_ref):
    x = x_ref[...]
    mean = jnp.mean(x, axis=-1, keepdims=True)
    xc = x - mean
    var = jnp.mean(xc * xc, axis=-1, keepdims=True)
    xln = xc / jnp.sqrt(var + 1e-5) * gamma_ref[0, :] + beta_ref[0, :]
    xln_ref[...] = xln

    t = jnp.dot(xln, wt_ref[...], preferred_element_type=jnp.float32)

    cb = cb_ref[...]
    cnorm = jnp.sum(cb * cb, axis=0, keepdims=True)
    scores = jnp.dot(t, cb, preferred_element_type=jnp.float32,
                     precision=jax.lax.Precision.HIGHEST)
    d2 = cnorm - 2.0 * scores

    iota = jax.lax.broadcasted_iota(jnp.int32, d2.shape, 1)
    sentinel = jnp.int32(_K)
    m1 = jnp.min(d2, axis=-1, keepdims=True)
    i1 = jnp.min(jnp.where(d2 == m1, iota, sentinel), axis=-1, keepdims=True)
    d2m = jnp.where(iota == i1, jnp.inf, d2)
    m2 = jnp.min(d2m, axis=-1, keepdims=True)
    i2 = jnp.min(jnp.where(d2m == m2, iota, sentinel), axis=-1, keepdims=True)

    oh1 = (iota == i1).astype(jnp.float32)
    oh2 = (iota == i2).astype(jnp.float32)
    cbt = cbt_ref[...]
    c1 = jnp.dot(oh1, cbt, preferred_element_type=jnp.float32,
                 precision=jax.lax.Precision.HIGHEST)
    c2 = jnp.dot(oh2, cbt, preferred_element_type=jnp.float32,
                 precision=jax.lax.Precision.HIGHEST)
    e1 = t - c1
    e2 = t - c2
    d1x = jnp.sqrt(jnp.sum(e1 * e1, axis=-1, keepdims=True))
    d2x = jnp.sqrt(jnp.sum(e2 * e2, axis=-1, keepdims=True))
    lab = jnp.where(d1x < d2x, i1,
                    jnp.where(d2x < d1x, i2, jnp.minimum(i1, i2)))
    lab_ref[...] = lab


def kernel(input_values, masking, gamma, beta, W, codebook):
    del masking
    x2d = input_values.reshape(_B * _T, _D)
    g2d = gamma.reshape(1, _D)
    b2d = beta.reshape(1, _D)
    wt = W.T
    cbt = codebook.T

    xln, lab = pl.pallas_call(
        _vq_body,
        grid=(_NB,),
        in_specs=[
            pl.BlockSpec((_TT, _D), lambda i: (i, 0)),
            pl.BlockSpec((1, _D), lambda i: (0, 0)),
            pl.BlockSpec((1, _D), lambda i: (0, 0)),
            pl.BlockSpec((_D, _Q), lambda i: (0, 0)),
            pl.BlockSpec((_Q, _K), lambda i: (0, 0)),
            pl.BlockSpec((_K, _Q), lambda i: (0, 0)),
        ],
        out_specs=[
            pl.BlockSpec((_TT, _D), lambda i: (i, 0)),
            pl.BlockSpec((_TT, 1), lambda i: (i, 0)),
        ],
        out_shape=[
            jax.ShapeDtypeStruct((_B * _T, _D), jnp.float32),
            jax.ShapeDtypeStruct((_B * _T, 1), jnp.int32),
        ],
    )(x2d, g2d, b2d, wt, codebook, cbt)

    return xln.reshape(_B, _T, _D), lab.reshape(_B, _T)


# speedup vs baseline: 6.0523x; 1.1741x over previous
import jax
import jax.numpy as jnp
from jax.experimental import pallas as pl

_B, _T, _D = 2, 2048, 768
_Q, _K = 64, 512
_TT = 512
_NB = (_B * _T) // _TT


def _vq_body(x_ref, gamma_ref, beta_ref, wt_ref, cb_ref, cbt_ref,
             xln_ref, lab_ref):
    x = x_ref[...]
    mean = jnp.mean(x, axis=-1, keepdims=True)
    xc = x - mean
    var = jnp.mean(xc * xc, axis=-1, keepdims=True)
    xln = xc / jnp.sqrt(var + 1e-5) * gamma_ref[0, :] + beta_ref[0, :]
    xln_ref[...] = xln

    t = jnp.dot(xln, wt_ref[...], preferred_element_type=jnp.float32)

    cb = cb_ref[...]
    cnorm = jnp.sum(cb * cb, axis=0, keepdims=True)
    t_hi = t.astype(jnp.bfloat16)
    t_lo = (t - t_hi.astype(jnp.float32)).astype(jnp.bfloat16)
    cb_hi = cb.astype(jnp.bfloat16)
    cb_lo = (cb - cb_hi.astype(jnp.float32)).astype(jnp.bfloat16)
    scores = (jnp.dot(t_hi, cb_hi, preferred_element_type=jnp.float32)
              + jnp.dot(t_hi, cb_lo, preferred_element_type=jnp.float32)
              + jnp.dot(t_lo, cb_hi, preferred_element_type=jnp.float32))
    d2 = cnorm - 2.0 * scores

    iota = jax.lax.broadcasted_iota(jnp.int32, d2.shape, 1)
    sentinel = jnp.int32(_K)
    m1 = jnp.min(d2, axis=-1, keepdims=True)
    i1 = jnp.min(jnp.where(d2 == m1, iota, sentinel), axis=-1, keepdims=True)
    d2m = jnp.where(iota == i1, jnp.inf, d2)
    m2 = jnp.min(d2m, axis=-1, keepdims=True)
    i2 = jnp.min(jnp.where(d2m == m2, iota, sentinel), axis=-1, keepdims=True)

    oh = jnp.concatenate([(iota == i1), (iota == i2)],
                         axis=0).astype(jnp.bfloat16)
    cbt = cbt_ref[...]
    hi = cbt.astype(jnp.bfloat16)
    r1 = cbt - hi.astype(jnp.float32)
    mid = r1.astype(jnp.bfloat16)
    lo = (r1 - mid.astype(jnp.float32)).astype(jnp.bfloat16)
    g = (jnp.dot(oh, hi, preferred_element_type=jnp.float32)
         + jnp.dot(oh, mid, preferred_element_type=jnp.float32)
         + jnp.dot(oh, lo, preferred_element_type=jnp.float32))
    c1 = g[:_TT]
    c2 = g[_TT:]
    e1 = t - c1
    e2 = t - c2
    d1x = jnp.sqrt(jnp.sum(e1 * e1, axis=-1, keepdims=True))
    d2x = jnp.sqrt(jnp.sum(e2 * e2, axis=-1, keepdims=True))
    lab = jnp.where(d1x < d2x, i1,
                    jnp.where(d2x < d1x, i2, jnp.minimum(i1, i2)))
    lab_ref[...] = lab


def kernel(input_values, masking, gamma, beta, W, codebook):
    del masking
    x2d = input_values.reshape(_B * _T, _D)
    g2d = gamma.reshape(1, _D)
    b2d = beta.reshape(1, _D)
    wt = W.T
    cbt = codebook.T

    xln, lab = pl.pallas_call(
        _vq_body,
        grid=(_NB,),
        in_specs=[
            pl.BlockSpec((_TT, _D), lambda i: (i, 0)),
            pl.BlockSpec((1, _D), lambda i: (0, 0)),
            pl.BlockSpec((1, _D), lambda i: (0, 0)),
            pl.BlockSpec((_D, _Q), lambda i: (0, 0)),
            pl.BlockSpec((_Q, _K), lambda i: (0, 0)),
            pl.BlockSpec((_K, _Q), lambda i: (0, 0)),
        ],
        out_specs=[
            pl.BlockSpec((_TT, _D), lambda i: (i, 0)),
            pl.BlockSpec((_TT, 1), lambda i: (i, 0)),
        ],
        out_shape=[
            jax.ShapeDtypeStruct((_B * _T, _D), jnp.float32),
            jax.ShapeDtypeStruct((_B * _T, 1), jnp.int32),
        ],
    )(x2d, g2d, b2d, wt, codebook, cbt)

    return xln.reshape(_B, _T, _D), lab.reshape(_B, _T)


# speedup vs baseline: 6.4421x; 1.0644x over previous
import jax
import jax.numpy as jnp
from jax.experimental import pallas as pl

_B, _T, _D = 2, 2048, 768
_Q, _K = 64, 512
_TT = 512


def _vq_body(x_ref, gamma_ref, beta_ref, wt_ref, cb_ref, cbt_ref,
             xln_ref, lab_ref):
    x = x_ref[...]
    mean = jnp.mean(x, axis=-1, keepdims=True)
    xc = x - mean
    var = jnp.mean(xc * xc, axis=-1, keepdims=True)
    xln = xc / jnp.sqrt(var + 1e-5) * gamma_ref[0, :] + beta_ref[0, :]
    xln_ref[...] = xln

    t = jnp.dot(xln, wt_ref[...], preferred_element_type=jnp.float32)

    cb = cb_ref[...]
    cnorm = jnp.sum(cb * cb, axis=0, keepdims=True)
    t_hi = t.astype(jnp.bfloat16)
    t_lo = (t - t_hi.astype(jnp.float32)).astype(jnp.bfloat16)
    cb_hi = cb.astype(jnp.bfloat16)
    cb_lo = (cb - cb_hi.astype(jnp.float32)).astype(jnp.bfloat16)
    tstack = jnp.concatenate([t_hi, t_hi, t_lo], axis=1)
    cbstack = jnp.concatenate([cb_hi, cb_lo, cb_hi], axis=0)
    d2 = cnorm - 2.0 * jnp.dot(tstack, cbstack,
                               preferred_element_type=jnp.float32)

    iota = jax.lax.broadcasted_iota(jnp.int32, d2.shape, 1)
    sentinel = jnp.int32(_K)
    m1 = jnp.min(d2, axis=-1, keepdims=True)
    i1 = jnp.min(jnp.where(d2 == m1, iota, sentinel), axis=-1, keepdims=True)
    d2m = jnp.where(iota == i1, jnp.inf, d2)
    m2 = jnp.min(d2m, axis=-1, keepdims=True)
    i2 = jnp.min(jnp.where(d2m == m2, iota, sentinel), axis=-1, keepdims=True)

    oh = jnp.concatenate([(iota == i1), (iota == i2)],
                         axis=0).astype(jnp.bfloat16)
    cbt = cbt_ref[...]
    hi = cbt.astype(jnp.bfloat16)
    r1 = cbt - hi.astype(jnp.float32)
    mid = r1.astype(jnp.bfloat16)
    lo = (r1 - mid.astype(jnp.float32)).astype(jnp.bfloat16)
    g = (jnp.dot(oh, hi, preferred_element_type=jnp.float32)
         + jnp.dot(oh, mid, preferred_element_type=jnp.float32)
         + jnp.dot(oh, lo, preferred_element_type=jnp.float32))
    c1 = g[:_TT]
    c2 = g[_TT:]
    e1 = t - c1
    e2 = t - c2
    d1x = jnp.sqrt(jnp.sum(e1 * e1, axis=-1, keepdims=True))
    d2x = jnp.sqrt(jnp.sum(e2 * e2, axis=-1, keepdims=True))
    lab = jnp.where(d1x < d2x, i1,
                    jnp.where(d2x < d1x, i2, jnp.minimum(i1, i2)))
    lab_ref[...] = lab


def kernel(input_values, masking, gamma, beta, W, codebook):
    del masking
    x2d = input_values.reshape(_B * _T, _D)
    g2d = gamma.reshape(1, _D)
    b2d = beta.reshape(1, _D)
    wt = W.T

    nb = (_B * _T) // _TT
    xln, lab = pl.pallas_call(
        _vq_body,
        grid=(nb,),
        in_specs=[
            pl.BlockSpec((_TT, _D), lambda i: (i, 0)),
            pl.BlockSpec((1, _D), lambda i: (0, 0)),
            pl.BlockSpec((1, _D), lambda i: (0, 0)),
            pl.BlockSpec((_D, _Q), lambda i: (0, 0)),
            pl.BlockSpec((_Q, _K), lambda i: (0, 0)),
            pl.BlockSpec((_K, _Q), lambda i: (0, 0)),
        ],
        out_specs=[
            pl.BlockSpec((_TT, _D), lambda i: (i, 0)),
            pl.BlockSpec((_TT, 1), lambda i: (i, 0)),
        ],
        out_shape=[
            jax.ShapeDtypeStruct((_B * _T, _D), jnp.float32),
            jax.ShapeDtypeStruct((_B * _T, 1), jnp.int32),
        ],
    )(x2d, g2d, b2d, wt, codebook, codebook.T)

    return xln.reshape(_B, _T, _D), lab.reshape(_B, _T)
